# Initial kernel scaffold; baseline (speedup 1.0000x reference)
import jax
import jax.numpy as jnp
from jax.experimental import pallas as pl


def kernel(inputs, indexes, labels, memory):
    raise NotImplementedError("write your pallas kernel here")


# speedup vs baseline: 13.4142x; 13.4142x over previous
import jax
import jax.numpy as jnp
from jax.experimental import pallas as pl
from jax.experimental.pallas import tpu as pltpu

B = 256
D = 2048
C = 8
CLS_PER_CAM = 1000
TOTAL_CLS = C * CLS_PER_CAM
NDATA = 16384
T = 0.07
HARD_NEG_K = 50
LOSS_WEIGHT = 0.5

_NEG_BIG = -1e9

_KEY_HI = 0x41800000
_KEY_LO = -0x41800001 - 1


def _sortable(x):
    b = jax.lax.bitcast_convert_type(x, jnp.int32)
    return jnp.where(b >= 0, b, b ^ jnp.int32(0x7FFFFFFF))


def _unsortable(k):
    b = jnp.where(k >= 0, k, k ^ jnp.int32(0x7FFFFFFF))
    return jax.lax.bitcast_convert_type(b, jnp.float32)


def _cap_kernel(x_ref, cams_ref, mapped_ref, mem_ref,
                intra_ref, inter_ref,
                xn_ref, t_ref, pos_ref, lse_ref):
    cc = pl.program_id(0)

    @pl.when(cc == 0)
    def _init():
        x = x_ref[...]
        inv = jax.lax.rsqrt(jnp.sum(x * x, axis=1, keepdims=True))
        xn_ref[...] = (x * inv).astype(jnp.bfloat16)
        pos_ref[...] = jnp.zeros((B, 1), jnp.float32)
        lse_ref[...] = jnp.zeros((B, 1), jnp.float32)

    @pl.when(cc < C)
    def _matmul_step():
        xn = xn_ref[...]
        blk = mem_ref[...].astype(jnp.bfloat16)
        s = jax.lax.dot_general(
            xn, blk, (((1,), (1,)), ((), ())),
            preferred_element_type=jnp.float32)
        t = s * (1.0 / T)
        cams = cams_ref[...]
        mapped = mapped_ref[...]
        row_in_cam = cams == cc
        col = jax.lax.broadcasted_iota(jnp.int32, (B, CLS_PER_CAM), 1)
        pos_mask = row_in_cam & (col == mapped)
        m = jnp.max(t, axis=1, keepdims=True)
        lse = m + jnp.log(jnp.sum(jnp.exp(t - m), axis=1, keepdims=True))
        pos = jnp.sum(jnp.where(pos_mask, t, 0.0), axis=1, keepdims=True)
        pos_ref[...] = jnp.where(row_in_cam, pos, pos_ref[...])
        lse_ref[...] = jnp.where(row_in_cam, lse, lse_ref[...])
        t_masked = jnp.where(pos_mask, _NEG_BIG, t)
        for k in range(C):
            @pl.when(cc == k)
            def _(k=k):
                t_ref[k] = t_masked

    @pl.when(cc == C)
    def _select_and_reduce():
        t = t_ref[...]
        pos = pos_ref[...]
        keys = _sortable(t)
        lo0 = jnp.full((1, B, 1), _KEY_LO, jnp.int32)
        hi0 = jnp.full((1, B, 1), _KEY_HI, jnp.int32)

        def body(_, carry):
            lo, hi = carry
            mid = (lo >> 1) + (hi >> 1) + (lo & hi & 1)
            cnt = jnp.sum((keys > mid).astype(jnp.int32), axis=(0, 2),
                          keepdims=True)
            ge = cnt >= HARD_NEG_K
            return jnp.where(ge, mid, lo), jnp.where(ge, hi, mid)

        lo, hi = jax.lax.fori_loop(0, 32, body, (lo0, hi0))
        tau = _unsortable(hi)
        pos3 = pos.reshape(1, B, 1)
        cnt_gt = jnp.sum((t > tau).astype(jnp.float32), axis=(0, 2),
                         keepdims=True)
        mrow = jnp.max(t, axis=(0, 2), keepdims=True)
        mref = jnp.maximum(mrow, pos3)
        s_top = jnp.sum(jnp.where(t > tau, jnp.exp(t - mref), 0.0),
                        axis=(0, 2), keepdims=True)
        s_fill = (jnp.float32(HARD_NEG_K) - cnt_gt) * jnp.exp(tau - mref)
        b_inter = (jnp.log(s_top + s_fill + jnp.exp(pos3 - mref))
                   + mref - pos3).reshape(B, 1)
        a_intra = lse_ref[...] - pos

        cams = cams_ref[...]
        li = jnp.zeros((1, 1), jnp.float32)
        le = jnp.zeros((1, 1), jnp.float32)
        for k in range(C):
            mask = cams == k
            n = jnp.sum(mask.astype(jnp.float32), axis=(0, 1), keepdims=True)
            denom = jnp.maximum(n, 1.0)
            sa = jnp.sum(jnp.where(mask, a_intra, 0.0), axis=(0, 1),
                         keepdims=True)
            sb = jnp.sum(jnp.where(mask, b_inter, 0.0), axis=(0, 1),
                         keepdims=True)
            present = n > 0.0
            li = li + jnp.where(present, sa / denom, 0.0)
            le = le + jnp.where(present, sb / denom, 0.0)
        intra_ref[...] = li
        inter_ref[...] = jnp.float32(LOSS_WEIGHT) * le


def _cap_pallas(inputs, cams, mapped, memory, interpret=False):
    return pl.pallas_call(
        _cap_kernel,
        grid=(C + 1,),
        in_specs=[
            pl.BlockSpec((B, D), lambda i: (0, 0)),
            pl.BlockSpec((B, 1), lambda i: (0, 0)),
            pl.BlockSpec((B, 1), lambda i: (0, 0)),
            pl.BlockSpec((CLS_PER_CAM, D),
                         lambda i: (jnp.minimum(i, C - 1), 0)),
        ],
        out_specs=[
            pl.BlockSpec((1, 1), lambda i: (0, 0)),
            pl.BlockSpec((1, 1), lambda i: (0, 0)),
        ],
        out_shape=[
            jax.ShapeDtypeStruct((1, 1), jnp.float32),
            jax.ShapeDtypeStruct((1, 1), jnp.float32),
        ],
        scratch_shapes=[
            pltpu.VMEM((B, D), jnp.bfloat16),
            pltpu.VMEM((C, B, CLS_PER_CAM), jnp.float32),
            pltpu.VMEM((B, 1), jnp.float32),
            pltpu.VMEM((B, 1), jnp.float32),
        ],
        interpret=interpret,
    )(inputs, cams, mapped, memory)


@jax.jit
def kernel(inputs, indexes, labels, memory):
    batch_labels = labels[indexes]
    cams = (batch_labels // CLS_PER_CAM).astype(jnp.int32).reshape(B, 1)
    mapped = (batch_labels % CLS_PER_CAM).astype(jnp.int32).reshape(B, 1)
    out = _cap_pallas(inputs, cams, mapped, memory)
    return (out[0][0, 0], out[1][0, 0])


# speedup vs baseline: 17.5031x; 1.3048x over previous
import jax
import jax.numpy as jnp
from jax.experimental import pallas as pl
from jax.experimental.pallas import tpu as pltpu

B = 256
D = 2048
C = 8
CLS_PER_CAM = 1000
TOTAL_CLS = C * CLS_PER_CAM
NDATA = 16384
T = 0.07
HARD_NEG_K = 50
LOSS_WEIGHT = 0.5

_NEG_BIG = -1e9

_KEY16_HI = 0x4180
_KEY16_LO = -0x4180 - 2


def _key16_to_f32(k):
    b = jnp.where(k >= 0, k, k ^ jnp.int32(0x7FFF))
    return jax.lax.bitcast_convert_type(b << 16, jnp.float32)


def _cap_kernel(x_ref, cams_ref, mapped_ref, mem_ref,
                intra_ref, inter_ref,
                xn_ref, t_ref, pos_ref, lse_ref):
    cc = pl.program_id(0)

    @pl.when(cc == 0)
    def _init():
        x = x_ref[...]
        inv = jax.lax.rsqrt(jnp.sum(x * x, axis=1, keepdims=True))
        xn_ref[...] = (x * inv).astype(jnp.bfloat16)
        pos_ref[...] = jnp.zeros((B, 1), jnp.float32)
        lse_ref[...] = jnp.zeros((B, 1), jnp.float32)

    @pl.when(cc < C)
    def _matmul_step():
        xn = xn_ref[...]
        blk = mem_ref[...].astype(jnp.bfloat16)
        s = jax.lax.dot_general(
            xn, blk, (((1,), (1,)), ((), ())),
            preferred_element_type=jnp.float32)
        t = s * (1.0 / T)
        cams = cams_ref[...]
        mapped = mapped_ref[...]
        row_in_cam = cams == cc
        col = jax.lax.broadcasted_iota(jnp.int32, (B, CLS_PER_CAM), 1)
        pos_mask = row_in_cam & (col == mapped)
        m = jnp.max(t, axis=1, keepdims=True)
        lse = m + jnp.log(jnp.sum(jnp.exp(t - m), axis=1, keepdims=True))
        pos = jnp.sum(jnp.where(pos_mask, t, 0.0), axis=1, keepdims=True)
        pos_ref[...] = jnp.where(row_in_cam, pos, pos_ref[...])
        lse_ref[...] = jnp.where(row_in_cam, lse, lse_ref[...])
        t_masked = jnp.where(pos_mask, _NEG_BIG, t)
        for k in range(C):
            @pl.when(cc == k)
            def _(k=k):
                t_ref[k] = t_masked

    @pl.when(cc == C)
    def _select_and_reduce():
        t = t_ref[...]
        pos = pos_ref[...]
        lo0 = jnp.full((1, B, 1), _KEY16_LO, jnp.int32)
        hi0 = jnp.full((1, B, 1), _KEY16_HI, jnp.int32)

        def body(_, carry):
            lo, hi = carry
            mid = (lo + hi) >> 1
            thr = _key16_to_f32(mid)
            cnt = jnp.sum(jnp.where(t > thr, 1.0, 0.0), axis=(0, 2),
                          keepdims=True)
            ge = cnt >= jnp.float32(HARD_NEG_K)
            return jnp.where(ge, mid, lo), jnp.where(ge, hi, mid)

        lo, hi = jax.lax.fori_loop(0, 16, body, (lo0, hi0))
        tau = _key16_to_f32(hi)
        tau_lo = _key16_to_f32(hi - 1)
        pos3 = pos.reshape(1, B, 1)
        mref = jnp.maximum(tau, pos3)
        gt = t > tau
        eq = (t > tau_lo) & jnp.logical_not(gt)
        e = jnp.exp(t - mref)
        cnt_gt = jnp.sum(jnp.where(gt, 1.0, 0.0), axis=(0, 2), keepdims=True)
        cnt_eq = jnp.sum(jnp.where(eq, 1.0, 0.0), axis=(0, 2), keepdims=True)
        s_top = jnp.sum(jnp.where(gt, e, 0.0), axis=(0, 2), keepdims=True)
        s_eq = jnp.sum(jnp.where(eq, e, 0.0), axis=(0, 2), keepdims=True)
        s_fill = (jnp.float32(HARD_NEG_K) - cnt_gt) * s_eq / cnt_eq
        b_inter = (jnp.log(s_top + s_fill + jnp.exp(pos3 - mref))
                   + mref - pos3).reshape(B, 1)
        a_intra = lse_ref[...] - pos

        cams = cams_ref[...]
        li = jnp.zeros((1, 1), jnp.float32)
        le = jnp.zeros((1, 1), jnp.float32)
        for k in range(C):
            mask = cams == k
            n = jnp.sum(mask.astype(jnp.float32), axis=(0, 1), keepdims=True)
            denom = jnp.maximum(n, 1.0)
            sa = jnp.sum(jnp.where(mask, a_intra, 0.0), axis=(0, 1),
                         keepdims=True)
            sb = jnp.sum(jnp.where(mask, b_inter, 0.0), axis=(0, 1),
                         keepdims=True)
            present = n > 0.0
            li = li + jnp.where(present, sa / denom, 0.0)
            le = le + jnp.where(present, sb / denom, 0.0)
        intra_ref[...] = li
        inter_ref[...] = jnp.float32(LOSS_WEIGHT) * le


def _cap_pallas(inputs, cams, mapped, memory, interpret=False):
    return pl.pallas_call(
        _cap_kernel,
        grid=(C + 1,),
        in_specs=[
            pl.BlockSpec((B, D), lambda i: (0, 0)),
            pl.BlockSpec((B, 1), lambda i: (0, 0)),
            pl.BlockSpec((B, 1), lambda i: (0, 0)),
            pl.BlockSpec((CLS_PER_CAM, D),
                         lambda i: (jnp.minimum(i, C - 1), 0)),
        ],
        out_specs=[
            pl.BlockSpec((1, 1), lambda i: (0, 0)),
            pl.BlockSpec((1, 1), lambda i: (0, 0)),
        ],
        out_shape=[
            jax.ShapeDtypeStruct((1, 1), jnp.float32),
            jax.ShapeDtypeStruct((1, 1), jnp.float32),
        ],
        scratch_shapes=[
            pltpu.VMEM((B, D), jnp.bfloat16),
            pltpu.VMEM((C, B, CLS_PER_CAM), jnp.float32),
            pltpu.VMEM((B, 1), jnp.float32),
            pltpu.VMEM((B, 1), jnp.float32),
        ],
        interpret=interpret,
    )(inputs, cams, mapped, memory)


@jax.jit
def kernel(inputs, indexes, labels, memory):
    batch_labels = labels[indexes]
    cams = (batch_labels // CLS_PER_CAM).astype(jnp.int32).reshape(B, 1)
    mapped = (batch_labels % CLS_PER_CAM).astype(jnp.int32).reshape(B, 1)
    out = _cap_pallas(inputs, cams, mapped, memory)
    return (out[0][0, 0], out[1][0, 0])
